# baseline (speedup 1.0000x reference)
import functools

import jax
import jax.numpy as jnp
from jax import lax
from jax.experimental import pallas as pl
from jax.experimental.pallas import tpu as pltpu
from jax.experimental.pallas import tpu_sc as plsc

N = 10000
E = 160000
D = 256
ED = 16
H = 8
DH = D // H
SCALE = DH ** (-0.5)

NC = 2
NS = 16
NW = NC * NS
L = 16
NPW = 313
EPW = E // NW
MAGIC = 26801
SHIFT = 23
EP = E + NW * 32
BLK = 40
NBLK = EPW // BLK
GC = 64
HPH = 4
DF = 128

BN = 1000

_mesh = plsc.VectorSubcoreMesh(core_axis_name="c", subcore_axis_name="s",
                               num_cores=NC, num_subcores=NS)


def _wid():
    return lax.axis_index("s") * NC + lax.axis_index("c")


def _qkv_body(x_ref, wq_ref, wk_ref, wv_ref, bq_ref, bk_ref, bv_ref,
              q_ref, k_ref, v_ref):
    x = x_ref[...]
    dn = (((1,), (1,)), ((), ()))
    q_ref[...] = jax.lax.dot_general(x, wq_ref[...], dn,
                                     preferred_element_type=jnp.float32) + bq_ref[...]
    k_ref[...] = jax.lax.dot_general(x, wk_ref[...], dn,
                                     preferred_element_type=jnp.float32) + bk_ref[...]
    v_ref[...] = jax.lax.dot_general(x, wv_ref[...], dn,
                                     preferred_element_type=jnp.float32) + bv_ref[...]


def _qkv(x, Wq, bq, Wk, bk, Wv, bv):
    grid = N // BN
    blk = pl.BlockSpec((BN, D), lambda i: (i, 0))
    full = pl.BlockSpec((D, D), lambda i: (0, 0))
    vec = pl.BlockSpec((1, D), lambda i: (0, 0))
    out = jax.ShapeDtypeStruct((N, D), jnp.float32)
    return pl.pallas_call(
        _qkv_body,
        grid=grid,
        in_specs=[blk, full, full, full, vec, vec, vec],
        out_specs=[blk, blk, blk],
        out_shape=[out, out, out],
    )(x, Wq, Wk, Wv, bq.reshape(1, D), bk.reshape(1, D), bv.reshape(1, D))


def _ew_body(ea_ref, we_ref, be_ref, ew_ref):
    dn = (((1,), (1,)), ((), ()))
    ew_ref[...] = jax.lax.dot_general(ea_ref[...], we_ref[...], dn,
                                      preferred_element_type=jnp.float32) + be_ref[...]


def _ew(edge_attr, We, be):
    BE = 8000
    return pl.pallas_call(
        _ew_body,
        grid=E // BE,
        in_specs=[pl.BlockSpec((BE, ED), lambda i: (i, 0)),
                  pl.BlockSpec((H, ED), lambda i: (0, 0)),
                  pl.BlockSpec((1, H), lambda i: (0, 0))],
        out_specs=pl.BlockSpec((BE, H), lambda i: (i, 0)),
        out_shape=jax.ShapeDtypeStruct((E, H), jnp.float32),
    )(edge_attr, We, be.reshape(1, H))


def _out_body(att_ref, wo_ref, bo_ref, g_ref, b_ref, o_ref):
    dn = (((1,), (1,)), ((), ()))
    o = jax.lax.dot_general(att_ref[...], wo_ref[...], dn,
                            preferred_element_type=jnp.float32) + bo_ref[...]
    mu = jnp.mean(o, axis=-1, keepdims=True)
    var = jnp.mean((o - mu) ** 2, axis=-1, keepdims=True)
    o_ref[...] = (o - mu) * jax.lax.rsqrt(var + 1e-5) * g_ref[...] + b_ref[...]


def _out_proj(att, Wo, bo, gamma, beta):
    return pl.pallas_call(
        _out_body,
        grid=N // BN,
        in_specs=[pl.BlockSpec((BN, D), lambda i: (i, 0)),
                  pl.BlockSpec((D, D), lambda i: (0, 0)),
                  pl.BlockSpec((1, D), lambda i: (0, 0)),
                  pl.BlockSpec((1, D), lambda i: (0, 0)),
                  pl.BlockSpec((1, D), lambda i: (0, 0))],
        out_specs=pl.BlockSpec((BN, D), lambda i: (i, 0)),
        out_shape=jax.ShapeDtypeStruct((N, D), jnp.float32),
    )(att, Wo, bo.reshape(1, D), gamma.reshape(1, D), beta.reshape(1, D))


def _hist_body(dst_hbm, hist_hbm, dst_v, hist_v, sem):
    wid = _wid()
    pltpu.async_copy(dst_hbm.at[pl.ds(pl.multiple_of(wid * EPW, 8), EPW)],
                     dst_v.at[pl.ds(0, EPW)], sem).wait()
    iota = lax.iota(jnp.int32, L)
    nchunks = (EPW + L - 1) // L
    zero = jnp.zeros((L,), jnp.int32)

    def chunk(i, carry):
        lo, hi = carry
        dvec = dst_v[pl.ds(i * L, L)]
        rem = EPW - i * L
        valid = iota < rem
        b = lax.shift_right_logical(dvec * MAGIC, SHIFT)
        b = jnp.where(valid, b, 9999)
        for lane in range(L):
            bs = jnp.full((L,), b[lane], jnp.int32)
            lo = lo + jnp.where(iota == bs, 1, 0)
            hi = hi + jnp.where(iota + L == bs, 1, 0)
        return lo, hi

    lo, hi = lax.fori_loop(0, nchunks, chunk, (zero, zero))
    hist_v[pl.ds(0, L)] = lo
    hist_v[pl.ds(L, L)] = hi
    pltpu.async_copy(hist_v, hist_hbm.at[pl.ds(pl.multiple_of(wid * NW, NW), NW)], sem).wait()


_hist = pl.kernel(
    _hist_body,
    out_type=jax.ShapeDtypeStruct((NW * NW,), jnp.int32),
    mesh=_mesh,
    compiler_params=pltpu.CompilerParams(needs_layout_passes=False),
    scratch_types=[pltpu.VMEM((EPW + L,), jnp.int32),
                   pltpu.VMEM((NW,), jnp.int32),
                   pltpu.SemaphoreType.DMA],
)


def _ceil32v(x):
    return lax.shift_left(lax.shift_right_logical(x + 31, 5), 5)


def _permute_body(dst_hbm, hist_hbm, peid_hbm, starts_hbm, totals_hbm,
                  grid_v, dst_v, cursor_v, idx_v, eid_v,
                  starts_v, totals_v, pad_idx, zero_v, sem, sem2):
    wid = _wid()
    iota = lax.iota(jnp.int32, L)
    z = jnp.zeros((L,), jnp.int32)
    pltpu.async_copy(hist_hbm, grid_v, sem).wait()
    pltpu.async_copy(dst_hbm.at[pl.ds(pl.multiple_of(wid * EPW, 8), EPW)],
                     dst_v.at[pl.ds(0, EPW)], sem).wait()

    def accum(w, carry):
        tl, th = carry
        return (tl + grid_v[pl.ds(w * NW, L)],
                th + grid_v[pl.ds(w * NW + L, L)])

    tot_lo, tot_hi = lax.fori_loop(0, NW, accum, (z, z))
    off_lo, off_hi = lax.fori_loop(0, wid, accum, (z, z))
    ceil_lo = _ceil32v(tot_lo)
    ceil_hi = _ceil32v(tot_hi)

    def _excl_prefix(vec):
        run = jnp.int32(0)
        exc = z
        for lane in range(L):
            exc = jnp.where(iota == lane, jnp.full((L,), run, jnp.int32), exc)
            run = run + vec[lane]
        return exc, run

    exc_lo, lo_sum = _excl_prefix(ceil_lo)
    exc_hi, hi_sum = _excl_prefix(ceil_hi)
    exc_hi = exc_hi + lo_sum
    total_end = lo_sum + hi_sum
    cursor_v[pl.ds(0, L)] = exc_lo + off_lo
    cursor_v[pl.ds(L, L)] = exc_hi + off_hi
    cursor_v[pl.ds(2 * L, L)] = z
    starts_v[pl.ds(0, L)] = exc_lo
    starts_v[pl.ds(L, L)] = exc_hi
    starts_v[pl.ds(2 * L, L)] = jnp.where(iota == 0, total_end, 0)
    totals_v[pl.ds(0, L)] = tot_lo
    totals_v[pl.ds(L, L)] = tot_hi
    totals_v[pl.ds(2 * L, L)] = z

    @pl.when(wid == 0)
    def _():
        zero_v[...] = z
        pltpu.async_copy(starts_v, starts_hbm, sem).wait()
        pltpu.async_copy(totals_v, totals_hbm, sem).wait()
        for b in range(NW):
            if b < L:
                base = exc_lo[b] + tot_lo[b]
                end = exc_lo[b] + ceil_lo[b]
            else:
                base = exc_hi[b - L] + tot_hi[b - L]
                end = exc_hi[b - L] + ceil_hi[b - L]
            for r in range(2):
                pos = base + (r * L) + iota
                safe = jnp.where(pos < end, pos, EP - 1)
                pad_idx[r] = safe
                pltpu.async_copy(zero_v, peid_hbm.at[pad_idx.at[r]],
                                 sem).wait()
        for r in range(2):
            pos = total_end + (r * L) + iota
            safe = jnp.where(pos < EP, pos, EP - 1)
            pad_idx[r] = safe
            pltpu.async_copy(zero_v, peid_hbm.at[pad_idx.at[r]],
                             sem).wait()

    def fill_eid(t, _):
        eid_v[pl.ds(t * L, L)] = wid * EPW + t * L + iota
        return 0
    lax.fori_loop(0, (EPW + L - 1) // L, fill_eid, 0)

    lane0 = iota == 0

    def place_chunk(c, _):
        dvec = dst_v[pl.ds(c * L, L)]
        valid = iota < (EPW - c * L)
        bvec = lax.shift_right_logical(dvec * MAGIC, SHIFT)
        bvec = jnp.where(valid, bvec, 0)
        nvalid = jnp.minimum(EPW - c * L, L)
        ib = c * L
        for lane in range(L):
            b = bvec[lane]
            i = ib + lane
            pos = cursor_v[pl.ds(b, L)][0]
            live = lane < nvalid
            m0 = jnp.logical_and(lane0, jnp.full((L,), live))
            plsc.store_scatter(cursor_v, [jnp.full((L,), b, jnp.int32)],
                               jnp.full((L,), pos + 1, jnp.int32), mask=m0)
            plsc.store_scatter(idx_v, [jnp.full((L,), lax.div(i, BLK), jnp.int32),
                                       jnp.full((L,), lax.rem(i, BLK), jnp.int32)],
                               jnp.full((L,), pos, jnp.int32), mask=m0)
        return 0

    lax.fori_loop(0, (EPW + L - 1) // L, place_chunk, 0)

    pending = []
    for j in range(NBLK):
        cp = pltpu.async_copy(eid_v.at[pl.ds(BLK * j, BLK)],
                              peid_hbm.at[idx_v.at[j]],
                              sem2)
        pending.append(cp)
    for cp in pending:
        cp.wait()


_permute = pl.kernel(
    _permute_body,
    out_type=[jax.ShapeDtypeStruct((EP,), jnp.int32),
              jax.ShapeDtypeStruct((NW + L,), jnp.int32),
              jax.ShapeDtypeStruct((NW + L,), jnp.int32)],
    mesh=_mesh,
    compiler_params=pltpu.CompilerParams(needs_layout_passes=False),
    scratch_types=[pltpu.VMEM((NW * NW,), jnp.int32),
                   pltpu.VMEM((EPW + L,), jnp.int32),
                   pltpu.VMEM((NW + L,), jnp.int32),
                   pltpu.VMEM((NBLK, BLK), jnp.int32),
                   pltpu.VMEM((EPW + L,), jnp.int32),
                   pltpu.VMEM((NW + L,), jnp.int32),
                   pltpu.VMEM((NW + L,), jnp.int32),
                   pltpu.VMEM((2, L), jnp.int32),
                   pltpu.VMEM((L,), jnp.int32),
                   pltpu.SemaphoreType.DMA,
                   pltpu.SemaphoreType.DMA],
)


def _main_body(qh_hbm, kh_hbm, vh_hbm, ew_hbm, src_hbm, dst_hbm,
               peid_hbm, starts_hbm, totals_hbm, att_hbm,
               num_v, den_v, peid_b, src_b, dst_b, qd_i, sv_i, dst_p,
               qr, kr, vr, ew_idx, ewg, s_buf,
               starts_v, totals_v, sp0, sp1, ssd0, ssd1, sr0, sr1, sem):
    wid = _wid()
    iota = lax.iota(jnp.int32, L)
    sp = (sp0, sp1)
    ssd = (ssd0, ssd1)
    sr = (sr0, sr1)

    pltpu.async_copy(starts_hbm, starts_v, sem).wait()
    pltpu.async_copy(totals_hbm, totals_v, sem).wait()

    zf = jnp.zeros((L,), jnp.float32)

    start = starts_v[pl.ds(wid, L)][0]
    tot = totals_v[pl.ds(wid, L)][0]
    nvol = lax.shift_right_logical(tot + (GC - 1), 6)

    def fire_peid(g, p):
        vbase = pl.multiple_of(start + g * GC, 32)
        pltpu.async_copy(peid_hbm.at[pl.ds(vbase, GC)], peid_b.at[p], sp[p])

    def wait_peid(p):
        pltpu.make_async_copy(peid_hbm.at[pl.ds(0, GC)], peid_b.at[p],
                              sp[p]).wait()

    def fire_sd(p, half):
        pltpu.async_copy(src_hbm.at[peid_b.at[p]], src_b.at[p], ssd[p])
        pltpu.async_copy(dst_hbm.at[peid_b.at[p]], dst_b.at[p], ssd[p])

    def wait_sd(p):
        pltpu.make_async_copy(src_hbm.at[pl.ds(0, GC)], src_b.at[p],
                              ssd[p]).wait()
        pltpu.make_async_copy(src_hbm.at[pl.ds(0, GC)], dst_b.at[p],
                              ssd[p]).wait()

    def fire_rows(p, half):
        hof = half * HPH
        for c4 in range(GC // L):
            pv = peid_b[p, pl.ds(c4 * L, L)] * H
            dv = dst_b[p, pl.ds(c4 * L, L)]
            sv = src_b[p, pl.ds(c4 * L, L)]
            qd_i[p, pl.ds(c4 * L, L)] = dv * 2 + half
            sv_i[p, pl.ds(c4 * L, L)] = sv * 2 + half
            for hh in range(HPH):
                pos = hh * GC + c4 * L
                ew_idx[p, pos // 128, pl.ds(pos % 128, L)] = pv + hof + hh
        pltpu.async_copy(qh_hbm.at[qd_i.at[p]], qr.at[p], sr[p])
        pltpu.async_copy(kh_hbm.at[sv_i.at[p]], kr.at[p], sr[p])
        pltpu.async_copy(vh_hbm.at[sv_i.at[p]], vr.at[p], sr[p])
        pltpu.async_copy(ew_hbm.at[ew_idx.at[p, 0]], ewg.at[p, 0], sr[p])
        pltpu.async_copy(ew_hbm.at[ew_idx.at[p, 1]], ewg.at[p, 1], sr[p])

    def wait_rows(p):
        pltpu.make_async_copy(qh_hbm.at[pl.ds(0, GC)], qr.at[p], sr[p]).wait()
        pltpu.make_async_copy(qh_hbm.at[pl.ds(0, GC)], kr.at[p], sr[p]).wait()
        pltpu.make_async_copy(qh_hbm.at[pl.ds(0, GC)], vr.at[p], sr[p]).wait()
        pltpu.make_async_copy(ew_hbm.at[pl.ds(0, 128)], ewg.at[p, 0],
                              sr[p]).wait()
        pltpu.make_async_copy(ew_hbm.at[pl.ds(0, 128)], ewg.at[p, 1],
                              sr[p]).wait()

    def compute_chunk(g, p, t):
        rem = tot - g * GC - t * L
        mask = iota < rem
        ri = iota + (t * L)
        dst_p[pl.ds(t * L, L)] = dst_b[p, pl.ds(t * L, L)]
        for hh in range(HPH):
            def dot_step(u, carry, hh=hh, ri=ri, p=p):
                a0, a1 = carry
                for w in range(16):
                    d = hh * DH + u * 16 + w
                    ci = jnp.full((L,), d, jnp.int32)
                    qd = plsc.load_gather(qr.at[p], [ri, ci])
                    kd = plsc.load_gather(kr.at[p], [ri, ci])
                    if w % 2 == 0:
                        a0 = a0 + qd * kd
                    else:
                        a1 = a1 + qd * kd
                return a0, a1
            acc0, acc1 = lax.fori_loop(0, DH // 16, dot_step, (zf, zf))
            acc = acc0 + acc1
            pos = hh * GC + t * L
            ewc = ewg[p, pos // 128, pl.ds(pos % 128, L)]
            s = jnp.exp(acc * SCALE + ewc)
            s = jnp.where(mask, s, 0.0)
            plsc.store_scatter(s_buf, [iota * L + hh], s)

        def edge_step(e, _, t=t, p=p):
            ea = t * L + e
            dl = dst_p[pl.ds(ea, L)][0] - wid * NPW
            dl = lax.max(lax.min(dl, NPW - 1), 0)
            srow = s_buf[pl.ds(e * L, L)]
            den_v[pl.ds(dl * L, L)] = den_v[pl.ds(dl * L, L)] + srow
            for hh in range(HPH):
                sv = jnp.full((L,), srow[hh], jnp.float32)
                for jj in range(2):
                    j = hh * 2 + jj
                    sl = pl.ds(dl * DF + j * L, L)
                    num_v[sl] = num_v[sl] + sv * vr[p, ea, pl.ds(j * L, L)]
            return 0

        lax.fori_loop(0, L, edge_step, 0)

    def run_half(half, _):
        def zero_num(i, _):
            num_v[pl.ds(i * L, L)] = zf
            return 0
        lax.fori_loop(0, NPW * DF // L, zero_num, 0)

        def zero_den(i, _):
            den_v[pl.ds(i * L, L)] = zf
            return 0
        lax.fori_loop(0, NPW, zero_den, 0)

        for r in range(L):
            s_buf[pl.ds(r * L, L)] = zf

        @pl.when(nvol > 0)
        def _():
            fire_peid(0, 0)
            wait_peid(0)
            fire_sd(0, half)
            wait_sd(0)
            fire_rows(0, half)

        @pl.when(nvol > 1)
        def _():
            fire_peid(1, 1)

        def pair(i, _):
            for pp in range(2):
                g = i * 2 + pp
                qq = 1 - pp

                @pl.when(g < nvol)
                def _(g=g, pp=pp, qq=qq):
                    @pl.when(g + 1 < nvol)
                    def _():
                        wait_peid(qq)
                        fire_sd(qq, half)
                    wait_rows(pp)
                    compute_chunk(g, pp, 0)
                    compute_chunk(g, pp, 1)

                    @pl.when(g + 1 < nvol)
                    def _():
                        wait_sd(qq)
                        fire_rows(qq, half)

                    @pl.when(g + 2 < nvol)
                    def _():
                        fire_peid(g + 2, pp)
                    compute_chunk(g, pp, 2)
                    compute_chunk(g, pp, 3)
            return 0

        lax.fori_loop(0, (nvol + 1) // 2, pair, 0)

        def norm(r, _):
            den = den_v[pl.ds(r * L, L)]
            inv = jnp.where(den > 0.0, 1.0 / den, 0.0)
            for hh in range(HPH):
                sv = jnp.full((L,), inv[hh], jnp.float32)
                for jj in range(2):
                    j = hh * 2 + jj
                    sl = pl.ds(r * DF + j * L, L)
                    num_v[sl] = num_v[sl] * sv
            return 0

        lax.fori_loop(0, NPW, norm, 0)

        off = pl.multiple_of(half * (NW * NPW * DF) + wid * (NPW * DF), 128)
        pltpu.async_copy(num_v, att_hbm.at[pl.ds(off, NPW * DF)], sem).wait()
        return 0

    lax.fori_loop(0, 2, run_half, 0)


_main = pl.kernel(
    _main_body,
    out_type=jax.ShapeDtypeStruct((2 * NW * NPW * DF,), jnp.float32),
    mesh=_mesh,
    compiler_params=pltpu.CompilerParams(needs_layout_passes=False),
    scratch_types=[pltpu.VMEM((NPW * DF,), jnp.float32),
                   pltpu.VMEM((NPW * L,), jnp.float32),
                   pltpu.VMEM((2, GC), jnp.int32),
                   pltpu.VMEM((2, GC), jnp.int32),
                   pltpu.VMEM((2, GC), jnp.int32),
                   pltpu.VMEM((2, GC), jnp.int32),
                   pltpu.VMEM((2, GC), jnp.int32),
                   pltpu.VMEM((GC + L,), jnp.int32),
                   pltpu.VMEM((2, GC, DF), jnp.float32),
                   pltpu.VMEM((2, GC, DF), jnp.float32),
                   pltpu.VMEM((2, GC, DF), jnp.float32),
                   pltpu.VMEM((2, 2, 128), jnp.int32),
                   pltpu.VMEM((2, 2, 128), jnp.float32),
                   pltpu.VMEM((L * L,), jnp.float32),
                   pltpu.VMEM((NW + L,), jnp.int32),
                   pltpu.VMEM((NW + L,), jnp.int32),
                   pltpu.SemaphoreType.DMA,
                   pltpu.SemaphoreType.DMA,
                   pltpu.SemaphoreType.DMA,
                   pltpu.SemaphoreType.DMA,
                   pltpu.SemaphoreType.DMA,
                   pltpu.SemaphoreType.DMA,
                   pltpu.SemaphoreType.DMA],
)


def kernel(x, edge_index, edge_attr, batch, Wq, bq, Wk, bk, Wv, bv,
           We, be, Wo, bo, gamma, beta):
    q, k, v = _qkv(x, Wq, bq, Wk, bk, Wv, bv)
    ew = _ew(edge_attr, We, be)
    src = edge_index[0]
    dst = edge_index[1]
    hist = _hist(dst)
    peid, starts, totals = _permute(dst, hist)
    att_flat = _main(q.reshape(2 * N, DF), k.reshape(2 * N, DF),
                     v.reshape(2 * N, DF), ew.reshape(E * H),
                     src, dst, peid, starts, totals)
    halves = att_flat.reshape(2, NW * NPW, DF)
    att = jnp.concatenate([halves[0], halves[1]], axis=1)[:N]
    return _out_proj(att, Wo, bo, gamma, beta)


# speedup vs baseline: 1.8620x; 1.8620x over previous
import functools

import jax
import jax.numpy as jnp
from jax import lax
from jax.experimental import pallas as pl
from jax.experimental.pallas import tpu as pltpu
from jax.experimental.pallas import tpu_sc as plsc

N = 10000
E = 160000
D = 256
ED = 16
H = 8
DH = D // H
SCALE = DH ** (-0.5)

NC = 2
NS = 16
NW = NC * NS
L = 16
NPW = 313
EPW = E // NW
MAGIC = 26801
SHIFT = 23
EP = E + NW * 32
BLK = 40
NBLK = EPW // BLK
GC = 64
HPH = 4
DF = 128

BN = 1000

_mesh = plsc.VectorSubcoreMesh(core_axis_name="c", subcore_axis_name="s",
                               num_cores=NC, num_subcores=NS)


def _wid():
    return lax.axis_index("s") * NC + lax.axis_index("c")


def _qkv_body(x_ref, wq_ref, wk_ref, wv_ref, bq_ref, bk_ref, bv_ref,
              q_ref, k_ref, v_ref):
    x = x_ref[...]
    dn = (((1,), (1,)), ((), ()))
    q_ref[...] = jax.lax.dot_general(x, wq_ref[...], dn,
                                     preferred_element_type=jnp.float32) + bq_ref[...]
    k_ref[...] = jax.lax.dot_general(x, wk_ref[...], dn,
                                     preferred_element_type=jnp.float32) + bk_ref[...]
    v_ref[...] = jax.lax.dot_general(x, wv_ref[...], dn,
                                     preferred_element_type=jnp.float32) + bv_ref[...]


def _qkv(x, Wq, bq, Wk, bk, Wv, bv):
    grid = N // BN
    blk = pl.BlockSpec((BN, D), lambda i: (i, 0))
    full = pl.BlockSpec((D, D), lambda i: (0, 0))
    vec = pl.BlockSpec((1, D), lambda i: (0, 0))
    out = jax.ShapeDtypeStruct((N, D), jnp.float32)
    return pl.pallas_call(
        _qkv_body,
        grid=grid,
        in_specs=[blk, full, full, full, vec, vec, vec],
        out_specs=[blk, blk, blk],
        out_shape=[out, out, out],
    )(x, Wq, Wk, Wv, bq.reshape(1, D), bk.reshape(1, D), bv.reshape(1, D))


def _ew_body(ea_ref, we_ref, be_ref, ew_ref):
    dn = (((1,), (1,)), ((), ()))
    ew_ref[...] = jax.lax.dot_general(ea_ref[...], we_ref[...], dn,
                                      preferred_element_type=jnp.float32) + be_ref[...]


def _ew(edge_attr, We, be):
    BE = 8000
    return pl.pallas_call(
        _ew_body,
        grid=E // BE,
        in_specs=[pl.BlockSpec((BE, ED), lambda i: (i, 0)),
                  pl.BlockSpec((H, ED), lambda i: (0, 0)),
                  pl.BlockSpec((1, H), lambda i: (0, 0))],
        out_specs=pl.BlockSpec((BE, H), lambda i: (i, 0)),
        out_shape=jax.ShapeDtypeStruct((E, H), jnp.float32),
    )(edge_attr, We, be.reshape(1, H))


def _out_body(att_ref, wo_ref, bo_ref, g_ref, b_ref, o_ref):
    dn = (((1,), (1,)), ((), ()))
    o = jax.lax.dot_general(att_ref[...], wo_ref[...], dn,
                            preferred_element_type=jnp.float32) + bo_ref[...]
    mu = jnp.mean(o, axis=-1, keepdims=True)
    var = jnp.mean((o - mu) ** 2, axis=-1, keepdims=True)
    o_ref[...] = (o - mu) * jax.lax.rsqrt(var + 1e-5) * g_ref[...] + b_ref[...]


def _out_proj(att, Wo, bo, gamma, beta):
    return pl.pallas_call(
        _out_body,
        grid=N // BN,
        in_specs=[pl.BlockSpec((BN, D), lambda i: (i, 0)),
                  pl.BlockSpec((D, D), lambda i: (0, 0)),
                  pl.BlockSpec((1, D), lambda i: (0, 0)),
                  pl.BlockSpec((1, D), lambda i: (0, 0)),
                  pl.BlockSpec((1, D), lambda i: (0, 0))],
        out_specs=pl.BlockSpec((BN, D), lambda i: (i, 0)),
        out_shape=jax.ShapeDtypeStruct((N, D), jnp.float32),
    )(att, Wo, bo.reshape(1, D), gamma.reshape(1, D), beta.reshape(1, D))


def _hist_body(dst_hbm, hist_hbm, dst_v, hist_v, sem):
    wid = _wid()
    pltpu.async_copy(dst_hbm.at[pl.ds(pl.multiple_of(wid * EPW, 8), EPW)],
                     dst_v.at[pl.ds(0, EPW)], sem).wait()
    iota = lax.iota(jnp.int32, L)
    nchunks = (EPW + L - 1) // L
    zero = jnp.zeros((L,), jnp.int32)

    def chunk(i, carry):
        lo, hi = carry
        dvec = dst_v[pl.ds(i * L, L)]
        rem = EPW - i * L
        valid = iota < rem
        b = lax.shift_right_logical(dvec * MAGIC, SHIFT)
        b = jnp.where(valid, b, 9999)
        for lane in range(L):
            bs = jnp.full((L,), b[lane], jnp.int32)
            lo = lo + jnp.where(iota == bs, 1, 0)
            hi = hi + jnp.where(iota + L == bs, 1, 0)
        return lo, hi

    lo, hi = lax.fori_loop(0, nchunks, chunk, (zero, zero))
    hist_v[pl.ds(0, L)] = lo
    hist_v[pl.ds(L, L)] = hi
    pltpu.async_copy(hist_v, hist_hbm.at[pl.ds(pl.multiple_of(wid * NW, NW), NW)], sem).wait()


_hist = pl.kernel(
    _hist_body,
    out_type=jax.ShapeDtypeStruct((NW * NW,), jnp.int32),
    mesh=_mesh,
    compiler_params=pltpu.CompilerParams(needs_layout_passes=False),
    scratch_types=[pltpu.VMEM((EPW + L,), jnp.int32),
                   pltpu.VMEM((NW,), jnp.int32),
                   pltpu.SemaphoreType.DMA],
)


def _ceil32v(x):
    return lax.shift_left(lax.shift_right_logical(x + 31, 5), 5)


def _permute_body(dst_hbm, hist_hbm, peid_hbm, starts_hbm, totals_hbm,
                  grid_v, dst_v, cursor_v, idx_v, eid_v,
                  starts_v, totals_v, pad_idx, zero_v, sem, sem2):
    wid = _wid()
    iota = lax.iota(jnp.int32, L)
    z = jnp.zeros((L,), jnp.int32)
    pltpu.async_copy(hist_hbm, grid_v, sem).wait()
    pltpu.async_copy(dst_hbm.at[pl.ds(pl.multiple_of(wid * EPW, 8), EPW)],
                     dst_v.at[pl.ds(0, EPW)], sem).wait()

    def accum(w, carry):
        tl, th = carry
        return (tl + grid_v[pl.ds(w * NW, L)],
                th + grid_v[pl.ds(w * NW + L, L)])

    tot_lo, tot_hi = lax.fori_loop(0, NW, accum, (z, z))
    off_lo, off_hi = lax.fori_loop(0, wid, accum, (z, z))
    ceil_lo = _ceil32v(tot_lo)
    ceil_hi = _ceil32v(tot_hi)

    def _excl_prefix(vec):
        run = jnp.int32(0)
        exc = z
        for lane in range(L):
            exc = jnp.where(iota == lane, jnp.full((L,), run, jnp.int32), exc)
            run = run + vec[lane]
        return exc, run

    exc_lo, lo_sum = _excl_prefix(ceil_lo)
    exc_hi, hi_sum = _excl_prefix(ceil_hi)
    exc_hi = exc_hi + lo_sum
    total_end = lo_sum + hi_sum
    cursor_v[pl.ds(0, L)] = exc_lo + off_lo
    cursor_v[pl.ds(L, L)] = exc_hi + off_hi
    cursor_v[pl.ds(2 * L, L)] = z
    starts_v[pl.ds(0, L)] = exc_lo
    starts_v[pl.ds(L, L)] = exc_hi
    starts_v[pl.ds(2 * L, L)] = jnp.where(iota == 0, total_end, 0)
    totals_v[pl.ds(0, L)] = tot_lo
    totals_v[pl.ds(L, L)] = tot_hi
    totals_v[pl.ds(2 * L, L)] = z

    @pl.when(wid == 0)
    def _():
        zero_v[...] = z
        pltpu.async_copy(starts_v, starts_hbm, sem).wait()
        pltpu.async_copy(totals_v, totals_hbm, sem).wait()
        for b in range(NW):
            if b < L:
                base = exc_lo[b] + tot_lo[b]
                end = exc_lo[b] + ceil_lo[b]
            else:
                base = exc_hi[b - L] + tot_hi[b - L]
                end = exc_hi[b - L] + ceil_hi[b - L]
            for r in range(2):
                pos = base + (r * L) + iota
                safe = jnp.where(pos < end, pos, EP - 1)
                pad_idx[r] = safe
                pltpu.async_copy(zero_v, peid_hbm.at[pad_idx.at[r]],
                                 sem).wait()
        for r in range(2):
            pos = total_end + (r * L) + iota
            safe = jnp.where(pos < EP, pos, EP - 1)
            pad_idx[r] = safe
            pltpu.async_copy(zero_v, peid_hbm.at[pad_idx.at[r]],
                             sem).wait()

    def fill_eid(t, _):
        eid_v[pl.ds(t * L, L)] = wid * EPW + t * L + iota
        return 0
    lax.fori_loop(0, (EPW + L - 1) // L, fill_eid, 0)

    lane0 = iota == 0

    def place_chunk(c, _):
        dvec = dst_v[pl.ds(c * L, L)]
        valid = iota < (EPW - c * L)
        bvec = lax.shift_right_logical(dvec * MAGIC, SHIFT)
        bvec = jnp.where(valid, bvec, 0)
        nvalid = jnp.minimum(EPW - c * L, L)
        ib = c * L
        for lane in range(L):
            b = bvec[lane]
            i = ib + lane
            pos = cursor_v[pl.ds(b, L)][0]
            live = lane < nvalid
            m0 = jnp.logical_and(lane0, jnp.full((L,), live))
            plsc.store_scatter(cursor_v, [jnp.full((L,), b, jnp.int32)],
                               jnp.full((L,), pos + 1, jnp.int32), mask=m0)
            plsc.store_scatter(idx_v, [jnp.full((L,), lax.div(i, BLK), jnp.int32),
                                       jnp.full((L,), lax.rem(i, BLK), jnp.int32)],
                               jnp.full((L,), pos, jnp.int32), mask=m0)
        return 0

    lax.fori_loop(0, (EPW + L - 1) // L, place_chunk, 0)

    pending = []
    for j in range(NBLK):
        cp = pltpu.async_copy(eid_v.at[pl.ds(BLK * j, BLK)],
                              peid_hbm.at[idx_v.at[j]],
                              sem2)
        pending.append(cp)
    for cp in pending:
        cp.wait()


_permute = pl.kernel(
    _permute_body,
    out_type=[jax.ShapeDtypeStruct((EP,), jnp.int32),
              jax.ShapeDtypeStruct((NW + L,), jnp.int32),
              jax.ShapeDtypeStruct((NW + L,), jnp.int32)],
    mesh=_mesh,
    compiler_params=pltpu.CompilerParams(needs_layout_passes=False),
    scratch_types=[pltpu.VMEM((NW * NW,), jnp.int32),
                   pltpu.VMEM((EPW + L,), jnp.int32),
                   pltpu.VMEM((NW + L,), jnp.int32),
                   pltpu.VMEM((NBLK, BLK), jnp.int32),
                   pltpu.VMEM((EPW + L,), jnp.int32),
                   pltpu.VMEM((NW + L,), jnp.int32),
                   pltpu.VMEM((NW + L,), jnp.int32),
                   pltpu.VMEM((2, L), jnp.int32),
                   pltpu.VMEM((L,), jnp.int32),
                   pltpu.SemaphoreType.DMA,
                   pltpu.SemaphoreType.DMA],
)


def _main_body(qh_hbm, kh_hbm, vh_hbm, ew_hbm, src_hbm, dst_hbm,
               peid_hbm, starts_hbm, totals_hbm, att_hbm,
               num_v, den_v, peid_b, src_b, dst_b, qd_i, sv_i, dst_p,
               qr, kr, vr, ew_idx, ewg,
               starts_v, totals_v, sp0, sp1, ssd0, ssd1, sr0, sr1, sem):
    wid = _wid()
    iota = lax.iota(jnp.int32, L)
    sp = (sp0, sp1)
    ssd = (ssd0, ssd1)
    sr = (sr0, sr1)

    pltpu.async_copy(starts_hbm, starts_v, sem).wait()
    pltpu.async_copy(totals_hbm, totals_v, sem).wait()

    zf = jnp.zeros((L,), jnp.float32)

    start = starts_v[pl.ds(wid, L)][0]
    tot = totals_v[pl.ds(wid, L)][0]
    nvol = lax.shift_right_logical(tot + (GC - 1), 6)

    def fire_peid(g, p):
        vbase = pl.multiple_of(start + g * GC, 32)
        pltpu.async_copy(peid_hbm.at[pl.ds(vbase, GC)], peid_b.at[p], sp[p])

    def wait_peid(p):
        pltpu.make_async_copy(peid_hbm.at[pl.ds(0, GC)], peid_b.at[p],
                              sp[p]).wait()

    def fire_sd(p, half):
        pltpu.async_copy(src_hbm.at[peid_b.at[p]], src_b.at[p], ssd[p])
        pltpu.async_copy(dst_hbm.at[peid_b.at[p]], dst_b.at[p], ssd[p])

    def wait_sd(p):
        pltpu.make_async_copy(src_hbm.at[pl.ds(0, GC)], src_b.at[p],
                              ssd[p]).wait()
        pltpu.make_async_copy(src_hbm.at[pl.ds(0, GC)], dst_b.at[p],
                              ssd[p]).wait()

    head_pat = jnp.where(iota < HPH, iota, 0)
    head_msk = iota < HPH

    def fire_rows(p, half):
        hof = half * HPH
        pvs = []
        for c4 in range(GC // L):
            pv = peid_b[p, pl.ds(c4 * L, L)]
            pvs.append(pv)
            dv = dst_b[p, pl.ds(c4 * L, L)]
            sv = src_b[p, pl.ds(c4 * L, L)]
            qd_i[p, pl.ds(c4 * L, L)] = dv * 2 + half
            sv_i[p, pl.ds(c4 * L, L)] = sv * 2 + half
        for e in range(GC):
            pv = pvs[e // L][e % L]
            base = jnp.full((L,), pv * H + hof, jnp.int32) + head_pat
            ew_idx[p, e // 8, pl.ds((e % 8) * L, L)] = base
        pltpu.async_copy(qh_hbm.at[qd_i.at[p]], qr.at[p], sr[p])
        pltpu.async_copy(kh_hbm.at[sv_i.at[p]], kr.at[p], sr[p])
        pltpu.async_copy(vh_hbm.at[sv_i.at[p]], vr.at[p], sr[p])
        for r in range(GC // 8):
            pltpu.async_copy(ew_hbm.at[ew_idx.at[p, r]],
                             ewg.at[p, pl.ds(r * 128, 128)], sr[p])

    def wait_rows(p):
        pltpu.make_async_copy(qh_hbm.at[pl.ds(0, GC)], qr.at[p], sr[p]).wait()
        pltpu.make_async_copy(qh_hbm.at[pl.ds(0, GC)], kr.at[p], sr[p]).wait()
        pltpu.make_async_copy(qh_hbm.at[pl.ds(0, GC)], vr.at[p], sr[p]).wait()
        for r in range(GC // 8):
            pltpu.make_async_copy(ew_hbm.at[pl.ds(0, 128)],
                                  ewg.at[p, pl.ds(r * 128, 128)],
                                  sr[p]).wait()

    def compute_edges(g, p, e_lo, e_hi):
        rem = tot - g * GC

        def estep(e, _, p=p):
            scs = []
            for hh in range(HPH):
                j0 = 2 * hh
                m0 = (qr[p, e, pl.ds(j0 * L, L)] *
                      kr[p, e, pl.ds(j0 * L, L)])
                m1 = (qr[p, e, pl.ds((j0 + 1) * L, L)] *
                      kr[p, e, pl.ds((j0 + 1) * L, L)])
                scs.append(jnp.sum(m0 + m1))
            sraw = zf
            for hh in range(HPH):
                sraw = jnp.where(iota == hh,
                                 jnp.full((L,), scs[hh], jnp.float32), sraw)
            ewr = ewg[p, pl.ds(e * L, L)]
            s = jnp.exp(sraw * SCALE + ewr)
            valid = jnp.logical_and(head_msk, jnp.full((L,), e < rem))
            s = jnp.where(valid, s, 0.0)
            dl = dst_p[pl.ds(e, L)][0] - wid * NPW
            dl = lax.max(lax.min(dl, NPW - 1), 0)
            den_v[pl.ds(dl * L, L)] = den_v[pl.ds(dl * L, L)] + s
            for hh in range(HPH):
                sv = jnp.full((L,), s[hh], jnp.float32)
                for jj in range(2):
                    j = hh * 2 + jj
                    sl = pl.ds(dl * DF + j * L, L)
                    num_v[sl] = num_v[sl] + sv * vr[p, e, pl.ds(j * L, L)]
            return 0

        lax.fori_loop(e_lo, e_hi, estep, 0)

    def run_half(half, _):
        def zero_num(i, _):
            num_v[pl.ds(i * L, L)] = zf
            return 0
        lax.fori_loop(0, NPW * DF // L, zero_num, 0)

        def zero_den(i, _):
            den_v[pl.ds(i * L, L)] = zf
            return 0
        lax.fori_loop(0, NPW, zero_den, 0)

        @pl.when(nvol > 0)
        def _():
            fire_peid(0, 0)
            wait_peid(0)
            fire_sd(0, half)
            wait_sd(0)
            fire_rows(0, half)

        @pl.when(nvol > 1)
        def _():
            fire_peid(1, 1)

        def pair(i, _):
            for pp in range(2):
                g = i * 2 + pp
                qq = 1 - pp

                @pl.when(g < nvol)
                def _(g=g, pp=pp, qq=qq):
                    @pl.when(g + 1 < nvol)
                    def _():
                        wait_peid(qq)
                        fire_sd(qq, half)
                    wait_rows(pp)
                    for c4 in range(GC // L):
                        dst_p[pl.ds(c4 * L, L)] = dst_b[pp, pl.ds(c4 * L, L)]
                    compute_edges(g, pp, 0, GC // 2)

                    @pl.when(g + 1 < nvol)
                    def _():
                        wait_sd(qq)
                        fire_rows(qq, half)

                    @pl.when(g + 2 < nvol)
                    def _():
                        fire_peid(g + 2, pp)
                    compute_edges(g, pp, GC // 2, GC)
            return 0

        lax.fori_loop(0, (nvol + 1) // 2, pair, 0)

        def norm(r, _):
            den = den_v[pl.ds(r * L, L)]
            inv = jnp.where(den > 0.0, 1.0 / den, 0.0)
            for hh in range(HPH):
                sv = jnp.full((L,), inv[hh], jnp.float32)
                for jj in range(2):
                    j = hh * 2 + jj
                    sl = pl.ds(r * DF + j * L, L)
                    num_v[sl] = num_v[sl] * sv
            return 0

        lax.fori_loop(0, NPW, norm, 0)

        off = pl.multiple_of(half * (NW * NPW * DF) + wid * (NPW * DF), 128)
        pltpu.async_copy(num_v, att_hbm.at[pl.ds(off, NPW * DF)], sem).wait()
        return 0

    lax.fori_loop(0, 2, run_half, 0)


_main = pl.kernel(
    _main_body,
    out_type=jax.ShapeDtypeStruct((2 * NW * NPW * DF,), jnp.float32),
    mesh=_mesh,
    compiler_params=pltpu.CompilerParams(needs_layout_passes=False),
    scratch_types=[pltpu.VMEM((NPW * DF,), jnp.float32),
                   pltpu.VMEM((NPW * L,), jnp.float32),
                   pltpu.VMEM((2, GC), jnp.int32),
                   pltpu.VMEM((2, GC), jnp.int32),
                   pltpu.VMEM((2, GC), jnp.int32),
                   pltpu.VMEM((2, GC), jnp.int32),
                   pltpu.VMEM((2, GC), jnp.int32),
                   pltpu.VMEM((GC + L,), jnp.int32),
                   pltpu.VMEM((2, GC, DF), jnp.float32),
                   pltpu.VMEM((2, GC, DF), jnp.float32),
                   pltpu.VMEM((2, GC, DF), jnp.float32),
                   pltpu.VMEM((2, GC // 8, 128), jnp.int32),
                   pltpu.VMEM((2, GC * L), jnp.float32),
                   pltpu.VMEM((NW + L,), jnp.int32),
                   pltpu.VMEM((NW + L,), jnp.int32),
                   pltpu.SemaphoreType.DMA,
                   pltpu.SemaphoreType.DMA,
                   pltpu.SemaphoreType.DMA,
                   pltpu.SemaphoreType.DMA,
                   pltpu.SemaphoreType.DMA,
                   pltpu.SemaphoreType.DMA,
                   pltpu.SemaphoreType.DMA],
)


def kernel(x, edge_index, edge_attr, batch, Wq, bq, Wk, bk, Wv, bv,
           We, be, Wo, bo, gamma, beta):
    q, k, v = _qkv(x, Wq, bq, Wk, bk, Wv, bv)
    ew = _ew(edge_attr, We, be)
    src = edge_index[0]
    dst = edge_index[1]
    hist = _hist(dst)
    peid, starts, totals = _permute(dst, hist)
    att_flat = _main(q.reshape(2 * N, DF), k.reshape(2 * N, DF),
                     v.reshape(2 * N, DF), ew.reshape(E * H),
                     src, dst, peid, starts, totals)
    halves = att_flat.reshape(2, NW * NPW, DF)
    att = jnp.concatenate([halves[0], halves[1]], axis=1)[:N]
    return _out_proj(att, Wo, bo, gamma, beta)
